# Initial kernel scaffold; baseline (speedup 1.0000x reference)
import jax
import jax.numpy as jnp
from jax.experimental import pallas as pl


def kernel(x, index):
    raise NotImplementedError("write your pallas kernel here")


# speedup vs baseline: 13.8999x; 13.8999x over previous
import functools

import jax
import jax.numpy as jnp
from jax import lax
from jax.experimental import pallas as pl
from jax.experimental.pallas import tpu as pltpu
from jax.experimental.pallas import tpu_sc as plsc

_NUM_CORES = 2
_NUM_SUBCORES = 16
_NUM_WORKERS = _NUM_CORES * _NUM_SUBCORES
_LANES = 16
_CHUNK = 128


@functools.partial(jax.jit, static_argnames=("batches", "rows_per_batch"))
def _selection_gather(x_flat, idx, *, batches, rows_per_batch):
    n_batches, k = idx.shape
    d = x_flat.shape[1]
    n_chunks = k // _CHUNK
    mesh = plsc.VectorSubcoreMesh(core_axis_name="c", subcore_axis_name="s")

    @functools.partial(
        pl.kernel,
        out_type=jax.ShapeDtypeStruct((n_batches, k, d), jnp.float32),
        mesh=mesh,
        scratch_types=[
            pltpu.VMEM((k,), jnp.int32),
            pltpu.VMEM((2, _CHUNK, d), jnp.float32),
            pltpu.SemaphoreType.DMA,
            pltpu.SemaphoreType.DMA,
            pltpu.SemaphoreType.DMA,
            pltpu.SemaphoreType.DMA,
        ],
    )
    def run(x_hbm, idx_hbm, out_hbm, idx_v, rows_v, g0, g1, w0, w1):
        gsem = (g0, g1)
        wsem = (w0, w1)
        b = lax.axis_index("s") * _NUM_CORES + lax.axis_index("c")

        pltpu.sync_copy(idx_hbm.at[b], idx_v)
        off = b * rows_per_batch

        def add_off(i, carry):
            sl = pl.ds(i * _LANES, _LANES)
            idx_v[sl] = idx_v[sl] + off
            return carry

        lax.fori_loop(0, k // _LANES, add_off, 0)

        gathers = {}
        writes = {}
        for c in range(n_chunks + 1):
            if c < n_chunks:
                buf = c % 2
                if c >= 2:
                    writes[c - 2].wait()
                h = pltpu.make_async_copy(
                    x_hbm.at[idx_v.at[pl.ds(c * _CHUNK, _CHUNK)]],
                    rows_v.at[buf],
                    gsem[buf],
                )
                h.start()
                gathers[c] = h
            if c >= 1:
                cc = c - 1
                buf = cc % 2
                gathers[cc].wait()
                hw = pltpu.make_async_copy(
                    rows_v.at[buf],
                    out_hbm.at[b, pl.ds(cc * _CHUNK, _CHUNK)],
                    wsem[buf],
                )
                hw.start()
                writes[cc] = hw
        writes[n_chunks - 2].wait()
        writes[n_chunks - 1].wait()

    return run(x_flat, idx)


def kernel(x, index):
    n_batches, n_rows, d = x.shape
    idx = index.astype(jnp.int32)
    x_flat = x.reshape(n_batches * n_rows, d)
    return _selection_gather(
        x_flat, idx, batches=n_batches, rows_per_batch=n_rows
    )


# speedup vs baseline: 14.1665x; 1.0192x over previous
import functools

import jax
import jax.numpy as jnp
from jax import lax
from jax.experimental import pallas as pl
from jax.experimental.pallas import tpu as pltpu
from jax.experimental.pallas import tpu_sc as plsc

_NUM_CORES = 2
_NUM_SUBCORES = 16
_NUM_WORKERS = _NUM_CORES * _NUM_SUBCORES
_LANES = 16
_CHUNK = 128
_NBUF = 4
_GDEPTH = 2


@functools.partial(jax.jit, static_argnames=("rows_per_batch",))
def _selection_gather(x_flat, idx, *, rows_per_batch):
    n_batches, k = idx.shape
    d = x_flat.shape[1]
    n_chunks = k // _CHUNK
    mesh = plsc.VectorSubcoreMesh(core_axis_name="c", subcore_axis_name="s")

    @functools.partial(
        pl.kernel,
        out_type=jax.ShapeDtypeStruct((n_batches, k, d), jnp.float32),
        mesh=mesh,
        scratch_types=[
            pltpu.VMEM((k,), jnp.int32),
            pltpu.VMEM((_NBUF, _CHUNK, d), jnp.float32),
            [pltpu.SemaphoreType.DMA] * _NBUF,
            [pltpu.SemaphoreType.DMA] * _NBUF,
        ],
    )
    def run(x_hbm, idx_hbm, out_hbm, idx_v, rows_v, gsem, wsem):
        b = lax.axis_index("s") * _NUM_CORES + lax.axis_index("c")
        off = b * rows_per_batch

        pltpu.sync_copy(idx_hbm.at[b], idx_v)

        def start_gather(c):
            h = pltpu.make_async_copy(
                x_hbm.at[idx_v.at[pl.ds(c * _CHUNK, _CHUNK)]],
                rows_v.at[c % _NBUF],
                gsem[c % _NBUF],
            )
            h.start()
            return h

        def start_write(c):
            h = pltpu.make_async_copy(
                rows_v.at[c % _NBUF],
                out_hbm.at[b, pl.ds(c * _CHUNK, _CHUNK)],
                wsem[c % _NBUF],
            )
            h.start()
            return h

        gathers, writes = {}, {}
        for c in range(n_chunks):
            for j in range(_CHUNK // _LANES):
                sl = pl.ds(c * _CHUNK + j * _LANES, _LANES)
                idx_v[sl] = idx_v[sl] + off
            if c >= _NBUF:
                writes[c - _NBUF].wait()
            gathers[c] = start_gather(c)
            if c >= _GDEPTH:
                cc = c - _GDEPTH
                gathers[cc].wait()
                writes[cc] = start_write(cc)
        for cc in range(n_chunks - _GDEPTH, n_chunks):
            gathers[cc].wait()
            writes[cc] = start_write(cc)
        for cc in range(n_chunks - _NBUF, n_chunks):
            writes[cc].wait()

    return run(x_flat, idx)


def kernel(x, index):
    n_batches, n_rows, d = x.shape
    idx = index.astype(jnp.int32)
    x_flat = x.reshape(n_batches * n_rows, d)
    return _selection_gather(x_flat, idx, rows_per_batch=n_rows)


# speedup vs baseline: 14.3019x; 1.0096x over previous
import functools

import jax
import jax.numpy as jnp
from jax import lax
from jax.experimental import pallas as pl
from jax.experimental.pallas import tpu as pltpu
from jax.experimental.pallas import tpu_sc as plsc

_NUM_CORES = 2
_NUM_SUBCORES = 16
_NUM_WORKERS = _NUM_CORES * _NUM_SUBCORES
_LANES = 16
_CHUNK = 128
_NBUF = 6
_GDEPTH = 3


@functools.partial(jax.jit, static_argnames=("rows_per_batch",))
def _selection_gather(x_flat, idx, *, rows_per_batch):
    n_batches, k = idx.shape
    d = x_flat.shape[1]
    n_chunks = k // _CHUNK
    mesh = plsc.VectorSubcoreMesh(core_axis_name="c", subcore_axis_name="s")

    @functools.partial(
        pl.kernel,
        out_type=jax.ShapeDtypeStruct((n_batches, k, d), jnp.float32),
        mesh=mesh,
        scratch_types=[
            pltpu.VMEM((k,), jnp.int32),
            pltpu.VMEM((_NBUF, _CHUNK, d), jnp.float32),
            [pltpu.SemaphoreType.DMA] * _NBUF,
            [pltpu.SemaphoreType.DMA] * _NBUF,
        ],
    )
    def run(x_hbm, idx_hbm, out_hbm, idx_v, rows_v, gsem, wsem):
        b = lax.axis_index("s") * _NUM_CORES + lax.axis_index("c")
        off = b * rows_per_batch

        pltpu.sync_copy(idx_hbm.at[b], idx_v)

        def start_gather(c):
            h = pltpu.make_async_copy(
                x_hbm.at[idx_v.at[pl.ds(c * _CHUNK, _CHUNK)]],
                rows_v.at[c % _NBUF],
                gsem[c % _NBUF],
            )
            h.start()
            return h

        def start_write(c):
            h = pltpu.make_async_copy(
                rows_v.at[c % _NBUF],
                out_hbm.at[b, pl.ds(c * _CHUNK, _CHUNK)],
                wsem[c % _NBUF],
            )
            h.start()
            return h

        gathers, writes = {}, {}
        for c in range(n_chunks):
            for j in range(_CHUNK // _LANES):
                sl = pl.ds(c * _CHUNK + j * _LANES, _LANES)
                idx_v[sl] = idx_v[sl] + off
            if c >= _NBUF:
                writes[c - _NBUF].wait()
            gathers[c] = start_gather(c)
            if c >= _GDEPTH:
                cc = c - _GDEPTH
                gathers[cc].wait()
                writes[cc] = start_write(cc)
        for cc in range(n_chunks - _GDEPTH, n_chunks):
            gathers[cc].wait()
            writes[cc] = start_write(cc)
        for cc in range(n_chunks - _NBUF, n_chunks):
            writes[cc].wait()

    return run(x_flat, idx)


def kernel(x, index):
    n_batches, n_rows, d = x.shape
    idx = index.astype(jnp.int32)
    x_flat = x.reshape(n_batches * n_rows, d)
    return _selection_gather(x_flat, idx, rows_per_batch=n_rows)


# speedup vs baseline: 14.3589x; 1.0040x over previous
import functools

import jax
import jax.numpy as jnp
from jax import lax
from jax.experimental import pallas as pl
from jax.experimental.pallas import tpu as pltpu
from jax.experimental.pallas import tpu_sc as plsc

_NUM_CORES = 2
_NUM_SUBCORES = 16
_NUM_WORKERS = _NUM_CORES * _NUM_SUBCORES
_LANES = 16
_CHUNK = 256
_NBUF = 3
_GDEPTH = 2


@functools.partial(jax.jit, static_argnames=("rows_per_batch",))
def _selection_gather(x_flat, idx, *, rows_per_batch):
    n_batches, k = idx.shape
    d = x_flat.shape[1]
    n_chunks = k // _CHUNK
    mesh = plsc.VectorSubcoreMesh(core_axis_name="c", subcore_axis_name="s")

    @functools.partial(
        pl.kernel,
        out_type=jax.ShapeDtypeStruct((n_batches, k, d), jnp.float32),
        mesh=mesh,
        scratch_types=[
            pltpu.VMEM((k,), jnp.int32),
            pltpu.VMEM((_NBUF, _CHUNK, d), jnp.float32),
            [pltpu.SemaphoreType.DMA] * _NBUF,
            [pltpu.SemaphoreType.DMA] * _NBUF,
        ],
    )
    def run(x_hbm, idx_hbm, out_hbm, idx_v, rows_v, gsem, wsem):
        b = lax.axis_index("s") * _NUM_CORES + lax.axis_index("c")
        off = b * rows_per_batch

        pltpu.sync_copy(idx_hbm.at[b], idx_v)

        def start_gather(c):
            h = pltpu.make_async_copy(
                x_hbm.at[idx_v.at[pl.ds(c * _CHUNK, _CHUNK)]],
                rows_v.at[c % _NBUF],
                gsem[c % _NBUF],
            )
            h.start()
            return h

        def start_write(c):
            h = pltpu.make_async_copy(
                rows_v.at[c % _NBUF],
                out_hbm.at[b, pl.ds(c * _CHUNK, _CHUNK)],
                wsem[c % _NBUF],
            )
            h.start()
            return h

        gathers, writes = {}, {}
        for c in range(n_chunks):
            for j in range(_CHUNK // _LANES):
                sl = pl.ds(c * _CHUNK + j * _LANES, _LANES)
                idx_v[sl] = idx_v[sl] + off
            if c >= _NBUF:
                writes[c - _NBUF].wait()
            gathers[c] = start_gather(c)
            if c >= _GDEPTH:
                cc = c - _GDEPTH
                gathers[cc].wait()
                writes[cc] = start_write(cc)
        for cc in range(n_chunks - _GDEPTH, n_chunks):
            gathers[cc].wait()
            writes[cc] = start_write(cc)
        for cc in range(n_chunks - _NBUF, n_chunks):
            writes[cc].wait()

    return run(x_flat, idx)


def kernel(x, index):
    n_batches, n_rows, d = x.shape
    idx = index.astype(jnp.int32)
    x_flat = x.reshape(n_batches * n_rows, d)
    return _selection_gather(x_flat, idx, rows_per_batch=n_rows)
